# Initial kernel scaffold; baseline (speedup 1.0000x reference)
import functools

import jax
import jax.numpy as jnp
from jax import lax
from jax.experimental import pallas as pl
from jax.experimental.pallas import tpu as pltpu
from jax.experimental.pallas import tpu_sc as plsc

B, H, W = 4, 512, 512
N = H * W
K = 1024
L = 16
NTILES = 16
IMGS_PER_CORE = B // 2
CHUNK = N // NTILES
NVEC = CHUNK // L
KROWS = K // L


def _sc_body(ndvi_hbm, slic_hbm, out_hbm,
             ndvi_v, slic_v, o0_v, o1_v, o2_v, hist_v, rowidx_v, sh_hist):
    c = lax.axis_index("c")
    s = lax.axis_index("s")
    ones_f = jnp.ones((L,), jnp.float32)
    zeros_f = jnp.zeros((L,), jnp.float32)
    iota = lax.iota(jnp.int32, L)

    for j in range(KROWS // L):
        rowidx_v[pl.ds(j * L, L)] = iota + j * L

    for bb in range(IMGS_PER_CORE):
        b = c * IMGS_PER_CORE + bb
        base_in = pl.multiple_of(b * N + s * CHUNK, CHUNK)
        pltpu.sync_copy(ndvi_hbm.at[pl.ds(base_in, CHUNK)], ndvi_v)
        pltpu.sync_copy(slic_hbm.at[pl.ds(base_in, CHUNK)], slic_v)

        def zero_body(j, _):
            hist_v[j, :] = zeros_f
            return 0
        lax.fori_loop(0, KROWS, zero_body, 0, unroll=8)

        @pl.when(s == 0)
        def _():
            pltpu.sync_copy(hist_v, sh_hist)

        def p1_body(i, _):
            nv = ndvi_v[pl.ds(i * L, L)]
            sv = slic_v[pl.ds(i * L, L)]
            m = nv > 0.0
            sid = jnp.where(m, sv, 0)
            plsc.addupdate_scatter(hist_v, [sid >> 4, sid & 15], ones_f, mask=m)
            return 0
        lax.fori_loop(0, NVEC, p1_body, 0, unroll=4)

        plsc.subcore_barrier()
        pltpu.sync_copy(hist_v, sh_hist.at[rowidx_v], add=True)
        plsc.subcore_barrier()
        pltpu.sync_copy(sh_hist, hist_v)

        def p2_body(i, _):
            nv = ndvi_v[pl.ds(i * L, L)]
            sv = slic_v[pl.ds(i * L, L)]
            m = nv > 0.0
            sid = jnp.where(m, sv, 0)
            cnt1 = plsc.load_gather(hist_v, [sid >> 4, sid & 15])
            apply = (cnt1 > 0.0) & (sid > 0)
            wm = jnp.where(m, 1, 0)
            lab = jnp.where(apply, 1, wm)
            o0_v[pl.ds(i * L, L)] = jnp.where(lab == 0, 1.0, 0.0)
            o1_v[pl.ds(i * L, L)] = jnp.where(lab == 1, 1.0, 0.0)
            o2_v[pl.ds(i * L, L)] = jnp.where(lab == 2, 1.0, 0.0)
            return 0
        lax.fori_loop(0, NVEC, p2_body, 0, unroll=4)

        base0 = pl.multiple_of((b * 3 + 0) * N + s * CHUNK, CHUNK)
        base1 = pl.multiple_of((b * 3 + 1) * N + s * CHUNK, CHUNK)
        base2 = pl.multiple_of((b * 3 + 2) * N + s * CHUNK, CHUNK)
        pltpu.sync_copy(o0_v, out_hbm.at[pl.ds(base0, CHUNK)])
        pltpu.sync_copy(o1_v, out_hbm.at[pl.ds(base1, CHUNK)])
        pltpu.sync_copy(o2_v, out_hbm.at[pl.ds(base2, CHUNK)])
        plsc.subcore_barrier()


@jax.jit
def _run(ndvi_flat, slic_flat):
    mesh = plsc.VectorSubcoreMesh(core_axis_name="c", subcore_axis_name="s")
    fn = functools.partial(
        pl.kernel,
        mesh=mesh,
        out_type=jax.ShapeDtypeStruct((B * 3 * N,), jnp.float32),
        scratch_types=[
            pltpu.VMEM((CHUNK,), jnp.float32),
            pltpu.VMEM((CHUNK,), jnp.int32),
            pltpu.VMEM((CHUNK,), jnp.float32),
            pltpu.VMEM((CHUNK,), jnp.float32),
            pltpu.VMEM((CHUNK,), jnp.float32),
            pltpu.VMEM((KROWS, L), jnp.float32),
            pltpu.VMEM((KROWS,), jnp.int32),
            pltpu.VMEM_SHARED((KROWS, L), jnp.float32),
        ],
    )(_sc_body)
    return fn(ndvi_flat, slic_flat)


def kernel(image, ndvi, slic):
    del image
    out = _run(ndvi.reshape(B * N), slic.reshape(B * N))
    return out.reshape(B, 3, H, W)


# speedup vs baseline: 165.9403x; 165.9403x over previous
import functools

import jax
import jax.numpy as jnp
from jax import lax
from jax.experimental import pallas as pl
from jax.experimental.pallas import tpu as pltpu
from jax.experimental.pallas import tpu_sc as plsc

B, H, W = 4, 512, 512
N = H * W
K = 1024
L = 16
NTILES = 16
IMGS_PER_CORE = B // 2
CHUNK = N // NTILES
NVEC = CHUNK // L
KROWS = K // L


def _sc_body(ndvi_hbm, slic_hbm, out_hbm,
             ndvi_v, slic_v, o0_v, o1_v, o2_v, hist_v, rowidx_v, sh_hist):
    c = lax.axis_index("c")
    s = lax.axis_index("s")
    ones_f = jnp.ones((L,), jnp.float32)
    zeros_f = jnp.zeros((L,), jnp.float32)
    iota = lax.iota(jnp.int32, L)

    for j in range(KROWS // L):
        rowidx_v[pl.ds(j * L, L)] = iota + j * L

    for bb in range(IMGS_PER_CORE):
        b = c * IMGS_PER_CORE + bb
        base_in = pl.multiple_of(b * N + s * CHUNK, CHUNK)
        pltpu.sync_copy(ndvi_hbm.at[pl.ds(base_in, CHUNK)], ndvi_v)
        pltpu.sync_copy(slic_hbm.at[pl.ds(base_in, CHUNK)], slic_v)

        def zero_body(j, _):
            hist_v[j, :] = zeros_f
            return 0
        lax.fori_loop(0, KROWS, zero_body, 0, unroll=8)

        @pl.when(s == 0)
        def _():
            pltpu.sync_copy(hist_v, sh_hist)

        def p1_body(i, _):
            nv = ndvi_v[pl.ds(i * L, L)]
            sv = slic_v[pl.ds(i * L, L)]
            m = nv > 0.0
            sid = jnp.where(m, sv, 0)
            plsc.addupdate_scatter(hist_v, [sid >> 4, sid & 15], ones_f, mask=m)
            return 0
        lax.fori_loop(0, NVEC, p1_body, 0, unroll=4)

        plsc.subcore_barrier()
        pltpu.sync_copy(hist_v, sh_hist.at[rowidx_v], add=True)
        plsc.subcore_barrier()
        pltpu.sync_copy(sh_hist, hist_v)

        def p2_body(i, _):
            nv = ndvi_v[pl.ds(i * L, L)]
            sv = slic_v[pl.ds(i * L, L)]
            m = nv > 0.0
            sid = jnp.where(m, sv, 0)
            cnt1 = plsc.load_gather(hist_v, [sid >> 4, sid & 15])
            apply = (cnt1 > 0.0) & (sid > 0)
            wm = jnp.where(m, 1, 0)
            lab = jnp.where(apply, 1, wm)
            o0_v[pl.ds(i * L, L)] = jnp.where(lab == 0, 1.0, 0.0)
            o1_v[pl.ds(i * L, L)] = jnp.where(lab == 1, 1.0, 0.0)
            o2_v[pl.ds(i * L, L)] = jnp.where(lab == 2, 1.0, 0.0)
            return 0
        lax.fori_loop(0, NVEC, p2_body, 0, unroll=4)

        base0 = pl.multiple_of((b * 3 + 0) * N + s * CHUNK, CHUNK)
        base1 = pl.multiple_of((b * 3 + 1) * N + s * CHUNK, CHUNK)
        base2 = pl.multiple_of((b * 3 + 2) * N + s * CHUNK, CHUNK)
        pltpu.sync_copy(o0_v, out_hbm.at[pl.ds(base0, CHUNK)])
        pltpu.sync_copy(o1_v, out_hbm.at[pl.ds(base1, CHUNK)])
        pltpu.sync_copy(o2_v, out_hbm.at[pl.ds(base2, CHUNK)])
        plsc.subcore_barrier()


@jax.jit
def _run(ndvi_flat, slic_flat):
    mesh = plsc.VectorSubcoreMesh(core_axis_name="c", subcore_axis_name="s")
    fn = functools.partial(
        pl.kernel,
        mesh=mesh,
        compiler_params=pltpu.CompilerParams(needs_layout_passes=False),
        out_type=jax.ShapeDtypeStruct((B * 3 * N,), jnp.float32),
        scratch_types=[
            pltpu.VMEM((CHUNK,), jnp.float32),
            pltpu.VMEM((CHUNK,), jnp.int32),
            pltpu.VMEM((CHUNK,), jnp.float32),
            pltpu.VMEM((CHUNK,), jnp.float32),
            pltpu.VMEM((CHUNK,), jnp.float32),
            pltpu.VMEM((KROWS, L), jnp.float32),
            pltpu.VMEM((KROWS,), jnp.int32),
            pltpu.VMEM_SHARED((KROWS, L), jnp.float32),
        ],
    )(_sc_body)
    return fn(ndvi_flat, slic_flat)


def kernel(image, ndvi, slic):
    del image
    out = _run(ndvi.reshape(B * N), slic.reshape(B * N))
    return out.reshape(B, 3, H, W)


# speedup vs baseline: 298.6412x; 1.7997x over previous
import functools

import jax
import jax.numpy as jnp
from jax import lax
from jax.experimental import pallas as pl
from jax.experimental.pallas import tpu as pltpu
from jax.experimental.pallas import tpu_sc as plsc

B, H, W = 4, 512, 512
N = H * W
K = 1024
L = 16
NTILES = 16
IMGS_PER_CORE = B // 2
CHUNK = N // NTILES
NVEC = CHUNK // L
KROWS = K // L


def _sc_body(ndvi_hbm, slic_hbm, out_hbm,
             ndvi_a, slic_a, ndvi_b, slic_b, o0_v, o1_v, o2_v,
             hist_v, rowidx_v, sh_hist, sem_in, sem_out):
    c = lax.axis_index("c")
    s = lax.axis_index("s")
    ones_f = jnp.ones((L,), jnp.float32)
    zeros_f = jnp.zeros((L,), jnp.float32)
    iota = lax.iota(jnp.int32, L)

    in_bufs = ((ndvi_a, slic_a), (ndvi_b, slic_b))
    in_handles = []
    for bb in range(IMGS_PER_CORE):
        b = c * IMGS_PER_CORE + bb
        base_in = pl.multiple_of(b * N + s * CHUNK, CHUNK)
        nv_ref, sv_ref = in_bufs[bb]
        in_handles.append((
            pltpu.async_copy(ndvi_hbm.at[pl.ds(base_in, CHUNK)], nv_ref, sem_in),
            pltpu.async_copy(slic_hbm.at[pl.ds(base_in, CHUNK)], sv_ref, sem_in),
        ))

    for j in range(KROWS // L):
        rowidx_v[pl.ds(j * L, L)] = iota + j * L

    @plsc.parallel_loop(0, CHUNK, step=L, unroll=8)
    def _(i):
        o2_v[pl.ds(i, L)] = zeros_f

    out_handles = ()
    for bb in range(IMGS_PER_CORE):
        b = c * IMGS_PER_CORE + bb
        nv_ref, sv_ref = in_bufs[bb]

        def zero_body(j, _):
            hist_v[j, :] = zeros_f
            return 0
        lax.fori_loop(0, KROWS, zero_body, 0, unroll=8)

        @pl.when(s == 0)
        def _():
            pltpu.sync_copy(hist_v, sh_hist)

        hn, hs = in_handles[bb]
        hn.wait()
        hs.wait()

        @plsc.parallel_loop(0, CHUNK, step=L, unroll=8)
        def _(i):
            nv = nv_ref[pl.ds(i, L)]
            sv = sv_ref[pl.ds(i, L)]
            m = nv > 0.0
            plsc.addupdate_scatter(
                hist_v, [sv >> 4, sv & 15], ones_f, mask=m)

        plsc.subcore_barrier()
        pltpu.sync_copy(hist_v, sh_hist.at[rowidx_v], add=True)
        plsc.subcore_barrier()
        pltpu.sync_copy(sh_hist, hist_v)
        plsc.subcore_barrier()

        for h in out_handles:
            h.wait()

        @plsc.parallel_loop(0, CHUNK, step=L, unroll=8)
        def _(i):
            nv = nv_ref[pl.ds(i, L)]
            sv = sv_ref[pl.ds(i, L)]
            m = nv > 0.0
            sid = jnp.where(m, sv, 0)
            cnt1 = plsc.load_gather(hist_v, [sid >> 4, sid & 15])
            apply = (cnt1 > 0.0) & (sid > 0)
            one = jnp.where(apply | m, 1.0, 0.0)
            o1_v[pl.ds(i, L)] = one
            o0_v[pl.ds(i, L)] = 1.0 - one

        base0 = pl.multiple_of((b * 3 + 0) * N + s * CHUNK, CHUNK)
        base1 = pl.multiple_of((b * 3 + 1) * N + s * CHUNK, CHUNK)
        base2 = pl.multiple_of((b * 3 + 2) * N + s * CHUNK, CHUNK)
        out_handles = (
            pltpu.async_copy(o0_v, out_hbm.at[pl.ds(base0, CHUNK)], sem_out),
            pltpu.async_copy(o1_v, out_hbm.at[pl.ds(base1, CHUNK)], sem_out),
            pltpu.async_copy(o2_v, out_hbm.at[pl.ds(base2, CHUNK)], sem_out),
        )

    for h in out_handles:
        h.wait()


@jax.jit
def _run(ndvi_flat, slic_flat):
    mesh = plsc.VectorSubcoreMesh(core_axis_name="c", subcore_axis_name="s")
    fn = functools.partial(
        pl.kernel,
        mesh=mesh,
        compiler_params=pltpu.CompilerParams(needs_layout_passes=False),
        out_type=jax.ShapeDtypeStruct((B * 3 * N,), jnp.float32),
        scratch_types=[
            pltpu.VMEM((CHUNK,), jnp.float32),
            pltpu.VMEM((CHUNK,), jnp.int32),
            pltpu.VMEM((CHUNK,), jnp.float32),
            pltpu.VMEM((CHUNK,), jnp.int32),
            pltpu.VMEM((CHUNK,), jnp.float32),
            pltpu.VMEM((CHUNK,), jnp.float32),
            pltpu.VMEM((CHUNK,), jnp.float32),
            pltpu.VMEM((KROWS, L), jnp.float32),
            pltpu.VMEM((KROWS,), jnp.int32),
            pltpu.VMEM_SHARED((KROWS, L), jnp.float32),
            pltpu.SemaphoreType.DMA,
            pltpu.SemaphoreType.DMA,
        ],
    )(_sc_body)
    return fn(ndvi_flat, slic_flat)


def kernel(image, ndvi, slic):
    del image
    out = _run(ndvi.reshape(B * N), slic.reshape(B * N))
    return out.reshape(B, 3, H, W)


# speedup vs baseline: 431.4984x; 1.4449x over previous
import functools

import jax
import jax.numpy as jnp
from jax import lax
from jax.experimental import pallas as pl
from jax.experimental.pallas import tpu as pltpu
from jax.experimental.pallas import tpu_sc as plsc

B, H, W = 4, 512, 512
K = 1024
L = 16
NTILES = 16
IMGS_PER_CORE = B // 2
ROWS = H // NTILES
CHUNK = ROWS * W
KROWS = K // L


def _sc_body(ndvi_hbm, slic_hbm, out_hbm,
             ndvi_a, slic_a, ndvi_b, slic_b, o0_v, o1_v, o2_v,
             hist_v, rowidx_v, sh_hist, sem_in, sem_out):
    c = lax.axis_index("c")
    s = lax.axis_index("s")
    ones_f = jnp.ones((L,), jnp.float32)
    zeros_f = jnp.zeros((L,), jnp.float32)
    iota = lax.iota(jnp.int32, L)
    r0 = s * ROWS

    in_bufs = ((ndvi_a, slic_a), (ndvi_b, slic_b))
    in_handles = []
    for bb in range(IMGS_PER_CORE):
        b = c * IMGS_PER_CORE + bb
        nv_ref, sv_ref = in_bufs[bb]
        in_handles.append((
            pltpu.async_copy(
                ndvi_hbm.at[b, pl.ds(r0, ROWS), :], nv_ref, sem_in),
            pltpu.async_copy(
                slic_hbm.at[b, pl.ds(r0, ROWS), :], sv_ref, sem_in),
        ))

    for j in range(KROWS // L):
        rowidx_v[pl.ds(j * L, L)] = iota + j * L

    @plsc.parallel_loop(0, CHUNK, step=L, unroll=8)
    def _(i):
        o2_v[i >> 9, pl.ds(i & (W - 1), L)] = zeros_f

    out_handles = ()
    for bb in range(IMGS_PER_CORE):
        b = c * IMGS_PER_CORE + bb
        nv_ref, sv_ref = in_bufs[bb]

        def zero_body(j, _):
            hist_v[j, :] = zeros_f
            return 0
        lax.fori_loop(0, KROWS, zero_body, 0, unroll=8)

        @pl.when(s == 0)
        def _():
            pltpu.sync_copy(hist_v, sh_hist)

        hn, hs = in_handles[bb]
        hn.wait()
        hs.wait()

        @plsc.parallel_loop(0, CHUNK, step=L, unroll=8)
        def _(i):
            r = i >> 9
            cc = i & (W - 1)
            nv = nv_ref[r, pl.ds(cc, L)]
            sv = sv_ref[r, pl.ds(cc, L)]
            m = nv > 0.0
            plsc.addupdate_scatter(
                hist_v, [sv >> 4, sv & 15], ones_f, mask=m)

        plsc.subcore_barrier()
        pltpu.sync_copy(hist_v, sh_hist.at[rowidx_v], add=True)
        plsc.subcore_barrier()
        pltpu.sync_copy(sh_hist, hist_v)
        plsc.subcore_barrier()

        for h in out_handles:
            h.wait()

        @plsc.parallel_loop(0, CHUNK, step=L, unroll=8)
        def _(i):
            r = i >> 9
            cc = i & (W - 1)
            nv = nv_ref[r, pl.ds(cc, L)]
            sv = sv_ref[r, pl.ds(cc, L)]
            m = nv > 0.0
            sid = jnp.where(m, sv, 0)
            cnt1 = plsc.load_gather(hist_v, [sid >> 4, sid & 15])
            apply = (cnt1 > 0.0) & (sid > 0)
            one = jnp.where(apply | m, 1.0, 0.0)
            o1_v[r, pl.ds(cc, L)] = one
            o0_v[r, pl.ds(cc, L)] = 1.0 - one

        out_handles = (
            pltpu.async_copy(o0_v, out_hbm.at[b, 0, pl.ds(r0, ROWS), :], sem_out),
            pltpu.async_copy(o1_v, out_hbm.at[b, 1, pl.ds(r0, ROWS), :], sem_out),
            pltpu.async_copy(o2_v, out_hbm.at[b, 2, pl.ds(r0, ROWS), :], sem_out),
        )

    for h in out_handles:
        h.wait()


@jax.jit
def _run(ndvi, slic):
    mesh = plsc.VectorSubcoreMesh(core_axis_name="c", subcore_axis_name="s")
    fn = functools.partial(
        pl.kernel,
        mesh=mesh,
        compiler_params=pltpu.CompilerParams(needs_layout_passes=False),
        out_type=jax.ShapeDtypeStruct((B, 3, H, W), jnp.float32),
        scratch_types=[
            pltpu.VMEM((ROWS, W), jnp.float32),
            pltpu.VMEM((ROWS, W), jnp.int32),
            pltpu.VMEM((ROWS, W), jnp.float32),
            pltpu.VMEM((ROWS, W), jnp.int32),
            pltpu.VMEM((ROWS, W), jnp.float32),
            pltpu.VMEM((ROWS, W), jnp.float32),
            pltpu.VMEM((ROWS, W), jnp.float32),
            pltpu.VMEM((KROWS, L), jnp.float32),
            pltpu.VMEM((KROWS,), jnp.int32),
            pltpu.VMEM_SHARED((KROWS, L), jnp.float32),
            pltpu.SemaphoreType.DMA,
            pltpu.SemaphoreType.DMA,
        ],
    )(_sc_body)
    return fn(ndvi, slic)


def kernel(image, ndvi, slic):
    del image
    return _run(ndvi, slic)
